# baseline (speedup 1.0000x reference)
import functools

import jax
import jax.numpy as jnp
from jax import lax
from jax.experimental import pallas as pl
from jax.experimental.pallas import tpu as pltpu
from jax.experimental.pallas import tpu_sc as plsc

VOCAB = 1000000
DIM = 64
B = 16384
NEG = 20
NSLAB = NEG + 1
SGRP = 3

NC = 2
NS = 16
NW = NC * NS
EPW = B // NW
C = 128
H = C // 2
NCHUNK = EPW // C


def _build_kernel():
    mesh = plsc.VectorSubcoreMesh(core_axis_name="c", subcore_axis_name="s")

    @functools.partial(
        pl.kernel,
        mesh=mesh,
        compiler_params=pltpu.CompilerParams(
            needs_layout_passes=False, use_tc_tiling_on_sc=False),
        out_type=jax.ShapeDtypeStruct((NSLAB, B), jnp.float32),
        scratch_types=[
            pltpu.VMEM((C,), jnp.int32),
            pltpu.VMEM((NSLAB, C), jnp.int32),
            pltpu.VMEM((C, DIM), jnp.float32),
            pltpu.VMEM((NSLAB, H, DIM), jnp.float32),
            pltpu.VMEM((NSLAB, C), jnp.float32),
            pltpu.SemaphoreType.DMA,
            pltpu.SemaphoreType.DMA((NSLAB,)),
        ],
    )
    def word2vec_sc(cidx_hbm, uidx_hbm, ctab_hbm, otab_hbm, out_hbm,
                    cidx_v, uidx_v, vc_v, u_v, sc_v, vc_sem, u_sems):
        wid = lax.axis_index("s") * NC + lax.axis_index("c")
        lanes = lax.iota(jnp.int32, 16)

        def compute_slabs(h, s_lo, refire):
            for k in range(SGRP):
                pltpu.make_async_copy(
                    otab_hbm.at[pl.ds(0, H)], u_v.at[s_lo + k],
                    u_sems.at[s_lo + k]).wait()
            for g in range(H // 16):
                e0 = g * 16
                accs = [jnp.zeros((16,), jnp.float32) for _ in range(SGRP)]
                for e in range(16):
                    ev = h * H + e0 + e
                    vc = [vc_v[ev, pl.ds(16 * j, 16)]
                          for j in range(DIM // 16)]
                    for k in range(SGRP):
                        s = s_lo + k
                        p = vc[0] * u_v[s, e0 + e, pl.ds(0, 16)]
                        for j in range(1, DIM // 16):
                            p = p + vc[j] * u_v[s, e0 + e, pl.ds(16 * j, 16)]
                        accs[k] = jnp.where(lanes == e, jnp.sum(p), accs[k])
                for k in range(SGRP):
                    sc_v[s_lo + k, pl.ds(h * H + e0, 16)] = accs[k]
            if refire:
                for k in range(SGRP):
                    s = s_lo + k
                    pltpu.async_copy(
                        otab_hbm.at[uidx_v.at[s, pl.ds(H, H)]],
                        u_v.at[s], u_sems.at[s])

        def chunk_body(ci, _):
            base = wid * EPW + ci * C
            pltpu.sync_copy(cidx_hbm.at[pl.ds(base, C)], cidx_v)
            pltpu.sync_copy(uidx_hbm.at[:, pl.ds(base, C)], uidx_v)

            vc_cp = pltpu.async_copy(ctab_hbm.at[cidx_v], vc_v, vc_sem)

            def fire0(s, _):
                pltpu.async_copy(otab_hbm.at[uidx_v.at[s, pl.ds(0, H)]],
                                 u_v.at[s], u_sems.at[s])
                return _
            lax.fori_loop(0, NSLAB, fire0, None)

            vc_cp.wait()

            def slabs0(i, _):
                compute_slabs(0, i * SGRP, refire=True)
                return _
            lax.fori_loop(0, NSLAB // SGRP, slabs0, None)

            def slabs1(i, _):
                compute_slabs(1, i * SGRP, refire=False)
                return _
            lax.fori_loop(0, NSLAB // SGRP, slabs1, None)

            pltpu.sync_copy(sc_v, out_hbm.at[:, pl.ds(base, C)])
            return _

        lax.fori_loop(0, NCHUNK, chunk_body, None)

    return word2vec_sc


_word2vec_sc = _build_kernel()


TBK = 512
NPAIR = -(-VOCAB // (2 * TBK))
VOCAB2 = 2 * NPAIR * TBK


def _relayout_tc(tab_t):
    grid = (NPAIR,)

    def body(a_ref, b_ref, out_ref):
        ya = jnp.transpose(a_ref[...])
        yb = jnp.transpose(b_ref[...])
        out_ref[...] = jnp.concatenate([ya, yb], axis=1)

    return pl.pallas_call(
        body,
        grid=grid,
        in_specs=[
            pl.BlockSpec((DIM, TBK), lambda i: (0, 2 * i)),
            pl.BlockSpec((DIM, TBK), lambda i: (0, 2 * i + 1)),
        ],
        out_specs=pl.BlockSpec((TBK, 2 * DIM), lambda i: (i, 0)),
        out_shape=jax.ShapeDtypeStruct((NPAIR * TBK, 2 * DIM), jnp.float32),
    )(tab_t, tab_t)


def _remap(v):
    return ((v >> 10) << 10) + 2 * (v & (TBK - 1)) + ((v >> 9) & 1)


def kernel(center_words, context_words, negative_samples, center_table, outside_table):
    uidx = _remap(jnp.concatenate(
        [context_words[None, :], negative_samples.T], axis=0))
    cidx = _remap(center_words)
    ctab = _relayout_tc(center_table.T).reshape(VOCAB2, DIM)
    otab = _relayout_tc(outside_table.T).reshape(VOCAB2, DIM)
    scores = _word2vec_sc(cidx, uidx, ctab, otab)
    return scores[0], scores[1:].T


# speedup vs baseline: 1.9251x; 1.9251x over previous
import functools

import jax
import jax.numpy as jnp
from jax import lax
from jax.experimental import pallas as pl
from jax.experimental.pallas import tpu as pltpu
from jax.experimental.pallas import tpu_sc as plsc

VOCAB = 1000000
DIM = 64
B = 16384
NEG = 20
NSLAB = NEG + 1
SGRP = 3

NC = 2
NS = 16
NW = NC * NS
EPW = B // NW
C = 128
H = C // 2
NCHUNK = EPW // C


def _build_kernel():
    mesh = plsc.VectorSubcoreMesh(core_axis_name="c", subcore_axis_name="s")

    @functools.partial(
        pl.kernel,
        mesh=mesh,
        compiler_params=pltpu.CompilerParams(
            needs_layout_passes=False, use_tc_tiling_on_sc=False),
        out_type=jax.ShapeDtypeStruct((NSLAB, B), jnp.float32),
        scratch_types=[
            pltpu.VMEM((C,), jnp.int32),
            pltpu.VMEM((NSLAB, C), jnp.int32),
            pltpu.VMEM((C, DIM), jnp.float32),
            pltpu.VMEM((NSLAB, H, DIM), jnp.float32),
            pltpu.VMEM((NSLAB, C), jnp.float32),
            pltpu.SemaphoreType.DMA,
            pltpu.SemaphoreType.DMA((NSLAB,)),
        ],
    )
    def word2vec_sc(cidx_hbm, uidx_hbm, ctab_hbm, otab_hbm, out_hbm,
                    cidx_v, uidx_v, vc_v, u_v, sc_v, vc_sem, u_sems):
        wid = lax.axis_index("s") * NC + lax.axis_index("c")
        lanes = lax.iota(jnp.int32, 16)

        def compute_slabs(h, s_lo, refire):
            for k in range(SGRP):
                pltpu.make_async_copy(
                    otab_hbm.at[pl.ds(0, H)], u_v.at[s_lo + k],
                    u_sems.at[s_lo + k]).wait()
            for g in range(H // 16):
                e0 = g * 16
                accs = [jnp.zeros((16,), jnp.float32) for _ in range(SGRP)]
                for e in range(16):
                    ev = h * H + e0 + e
                    vc = [vc_v[ev, pl.ds(16 * j, 16)]
                          for j in range(DIM // 16)]
                    for k in range(SGRP):
                        s = s_lo + k
                        p = vc[0] * u_v[s, e0 + e, pl.ds(0, 16)]
                        for j in range(1, DIM // 16):
                            p = p + vc[j] * u_v[s, e0 + e, pl.ds(16 * j, 16)]
                        accs[k] = jnp.where(lanes == e, jnp.sum(p), accs[k])
                for k in range(SGRP):
                    sc_v[s_lo + k, pl.ds(h * H + e0, 16)] = accs[k]
            if refire:
                for k in range(SGRP):
                    s = s_lo + k
                    pltpu.async_copy(
                        otab_hbm.at[uidx_v.at[s, pl.ds(H, H)]],
                        u_v.at[s], u_sems.at[s])

        def chunk_body(ci, _):
            base = wid * EPW + ci * C
            pltpu.sync_copy(cidx_hbm.at[pl.ds(base, C)], cidx_v)
            pltpu.sync_copy(uidx_hbm.at[:, pl.ds(base, C)], uidx_v)

            vc_cp = pltpu.async_copy(ctab_hbm.at[cidx_v], vc_v, vc_sem)

            def fire0(s, _):
                pltpu.async_copy(otab_hbm.at[uidx_v.at[s, pl.ds(0, H)]],
                                 u_v.at[s], u_sems.at[s])
                return _
            lax.fori_loop(0, NSLAB, fire0, None)

            vc_cp.wait()

            def slabs0(i, _):
                compute_slabs(0, i * SGRP, refire=True)
                return _
            lax.fori_loop(0, NSLAB // SGRP, slabs0, None)

            def slabs1(i, _):
                compute_slabs(1, i * SGRP, refire=False)
                return _
            lax.fori_loop(0, NSLAB // SGRP, slabs1, None)

            pltpu.sync_copy(sc_v, out_hbm.at[:, pl.ds(base, C)])
            return _

        lax.fori_loop(0, NCHUNK, chunk_body, None)

    return word2vec_sc


_word2vec_sc = _build_kernel()


TBK = 2048
TSH = TBK.bit_length() - 1
NPAIR = -(-VOCAB // (2 * TBK))
NB = -(-VOCAB // TBK)
VOCAB2 = 2 * NPAIR * TBK


def _relayout_tc(tab_t):
    grid = (NPAIR,)

    def body(a_ref, b_ref, out_ref):
        ya = jnp.transpose(a_ref[...])
        yb = jnp.transpose(b_ref[...])
        out_ref[...] = jnp.concatenate([ya, yb], axis=1)

    return pl.pallas_call(
        body,
        grid=grid,
        in_specs=[
            pl.BlockSpec((DIM, TBK), lambda i: (0, 2 * i)),
            pl.BlockSpec((DIM, TBK),
                         lambda i: (0, jnp.minimum(2 * i + 1, NB - 1))),
        ],
        out_specs=pl.BlockSpec((TBK, 2 * DIM), lambda i: (i, 0)),
        out_shape=jax.ShapeDtypeStruct((NPAIR * TBK, 2 * DIM), jnp.float32),
    )(tab_t, tab_t)


def _remap(v):
    return (((v >> (TSH + 1)) << (TSH + 1)) + 2 * (v & (TBK - 1))
            + ((v >> TSH) & 1))


def kernel(center_words, context_words, negative_samples, center_table, outside_table):
    uidx = _remap(jnp.concatenate(
        [context_words[None, :], negative_samples.T], axis=0))
    cidx = _remap(center_words)
    ctab = _relayout_tc(center_table.T).reshape(VOCAB2, DIM)
    otab = _relayout_tc(outside_table.T).reshape(VOCAB2, DIM)
    scores = _word2vec_sc(cidx, uidx, ctab, otab)
    return scores[0], scores[1:].T


# speedup vs baseline: 2.3308x; 1.2107x over previous
import functools

import jax
import jax.numpy as jnp
from jax import lax
from jax.experimental import pallas as pl
from jax.experimental.pallas import tpu as pltpu
from jax.experimental.pallas import tpu_sc as plsc

VOCAB = 1000000
DIM = 64
B = 16384
NEG = 20
NSLAB = NEG + 1
SGRP = 3

NC = 2
NS = 16
NW = NC * NS
EPW = B // NW
C = 128
H = C // 2
NCHUNK = EPW // C


def _build_kernel():
    mesh = plsc.VectorSubcoreMesh(core_axis_name="c", subcore_axis_name="s")

    @functools.partial(
        pl.kernel,
        mesh=mesh,
        compiler_params=pltpu.CompilerParams(
            needs_layout_passes=False, use_tc_tiling_on_sc=False),
        out_type=jax.ShapeDtypeStruct((NSLAB, B), jnp.float32),
        scratch_types=[
            pltpu.VMEM((C,), jnp.int32),
            pltpu.VMEM((NSLAB, C), jnp.int32),
            pltpu.VMEM((C, DIM), jnp.float32),
            pltpu.VMEM((NSLAB, H, DIM), jnp.float32),
            pltpu.VMEM((NSLAB, C), jnp.float32),
            pltpu.SemaphoreType.DMA,
            pltpu.SemaphoreType.DMA((NSLAB,)),
        ],
    )
    def word2vec_sc(cidx_hbm, uidx_hbm, ctab_hbm, otab_hbm, out_hbm,
                    cidx_v, uidx_v, vc_v, u_v, sc_v, vc_sem, u_sems):
        wid = lax.axis_index("s") * NC + lax.axis_index("c")
        lanes = lax.iota(jnp.int32, 16)

        def compute_slabs(h, s_lo, refire):
            for k in range(SGRP):
                pltpu.make_async_copy(
                    otab_hbm.at[pl.ds(0, H)], u_v.at[s_lo + k],
                    u_sems.at[s_lo + k]).wait()
            for g in range(H // 16):
                e0 = g * 16
                accs = [jnp.zeros((16,), jnp.float32) for _ in range(SGRP)]
                for e in range(16):
                    ev = h * H + e0 + e
                    vc = [vc_v[ev, pl.ds(16 * j, 16)]
                          for j in range(DIM // 16)]
                    for k in range(SGRP):
                        s = s_lo + k
                        p = vc[0] * u_v[s, e0 + e, pl.ds(0, 16)]
                        for j in range(1, DIM // 16):
                            p = p + vc[j] * u_v[s, e0 + e, pl.ds(16 * j, 16)]
                        accs[k] = jnp.where(lanes == e, jnp.sum(p), accs[k])
                for k in range(SGRP):
                    sc_v[s_lo + k, pl.ds(h * H + e0, 16)] = accs[k]
            if refire:
                for k in range(SGRP):
                    s = s_lo + k
                    pltpu.async_copy(
                        otab_hbm.at[uidx_v.at[s, pl.ds(H, H)]],
                        u_v.at[s], u_sems.at[s])

        def chunk_body(ci, _):
            base = wid * EPW + ci * C
            pltpu.sync_copy(cidx_hbm.at[pl.ds(base, C)], cidx_v)
            pltpu.sync_copy(uidx_hbm.at[:, pl.ds(base, C)], uidx_v)

            vc_cp = pltpu.async_copy(ctab_hbm.at[cidx_v], vc_v, vc_sem)

            def fire0(s, _):
                pltpu.async_copy(otab_hbm.at[uidx_v.at[s, pl.ds(0, H)]],
                                 u_v.at[s], u_sems.at[s])
                return _
            lax.fori_loop(0, NSLAB, fire0, None)

            vc_cp.wait()

            def slabs0(i, _):
                compute_slabs(0, i * SGRP, refire=True)
                return _
            lax.fori_loop(0, NSLAB // SGRP, slabs0, None)

            def slabs1(i, _):
                compute_slabs(1, i * SGRP, refire=False)
                return _
            lax.fori_loop(0, NSLAB // SGRP, slabs1, None)

            pltpu.sync_copy(sc_v, out_hbm.at[:, pl.ds(base, C)])
            return _

        lax.fori_loop(0, NCHUNK, chunk_body, None)

    return word2vec_sc


_word2vec_sc = _build_kernel()


TBK = 4096
TSH = TBK.bit_length() - 1
NPAIR = -(-VOCAB // (2 * TBK))
NB = -(-VOCAB // TBK)
VOCAB2 = 2 * NPAIR * TBK


def _relayout_tc(tab_t):
    grid = (NPAIR,)

    def body(a_ref, b_ref, out_ref):
        ya = jnp.transpose(a_ref[...])
        yb = jnp.transpose(b_ref[...])
        out_ref[...] = jnp.concatenate([ya, yb], axis=1)

    return pl.pallas_call(
        body,
        grid=grid,
        in_specs=[
            pl.BlockSpec((DIM, TBK), lambda i: (0, 2 * i)),
            pl.BlockSpec((DIM, TBK),
                         lambda i: (0, jnp.minimum(2 * i + 1, NB - 1))),
        ],
        out_specs=pl.BlockSpec((TBK, 2 * DIM), lambda i: (i, 0)),
        out_shape=jax.ShapeDtypeStruct((NPAIR * TBK, 2 * DIM), jnp.float32),
    )(tab_t, tab_t)


def _remap(v):
    return (((v >> (TSH + 1)) << (TSH + 1)) + 2 * (v & (TBK - 1))
            + ((v >> TSH) & 1))


def kernel(center_words, context_words, negative_samples, center_table, outside_table):
    uidx = _remap(jnp.concatenate(
        [context_words[None, :], negative_samples.T], axis=0))
    cidx = _remap(center_words)
    ctab = _relayout_tc(center_table.T).reshape(VOCAB2, DIM)
    otab = _relayout_tc(outside_table.T).reshape(VOCAB2, DIM)
    scores = _word2vec_sc(cidx, uidx, ctab, otab)
    return scores[0], scores[1:].T
